# baseline (speedup 1.0000x reference)
import functools

import jax
import jax.numpy as jnp
from jax import lax
from jax.experimental import pallas as pl
from jax.experimental.pallas import tpu as pltpu
import jax.experimental.pallas.tpu_sc as plsc

EPS = 1e-5
LANES = 16

_GATHER_DNUMS = lax.GatherDimensionNumbers(
    offset_dims=(), collapsed_slice_dims=(0,), start_index_map=(0,))


def _lane_total(v):
    iota = lax.iota(jnp.int32, LANES)
    for sh in (8, 4, 2, 1):
        idx = jnp.bitwise_xor(iota, sh)
        v = v + lax.gather(v, idx[:, None], _GATHER_DNUMS, (1,),
                           mode=lax.GatherScatterMode.PROMISE_IN_BOUNDS)
    return v


def _ln_rows(rows_v, b, pos_v, wv, bv, base, chunk, l, d, unroll=4,
             loop=None):
    ng = d // LANES
    if loop is None:
        loop = plsc.parallel_loop(0, chunk, unroll=unroll)

    @loop
    def _row(r):
        p = pl.multiple_of(lax.rem(base + r, l) * d, d)
        e = [rows_v[b, r, pl.ds(LANES * j, LANES)]
             + pos_v[pl.ds(p + LANES * j, LANES)] for j in range(ng)]
        s01, s23 = e[0] + e[1], e[2] + e[3]
        s45, s67 = e[4] + e[5], e[6] + e[7]
        s = (s01 + s23) + (s45 + s67)
        q = [ej * ej for ej in e]
        q01, q23 = q[0] + q[1], q[2] + q[3]
        q45, q67 = q[4] + q[5], q[6] + q[7]
        qs = (q01 + q23) + (q45 + q67)
        tv = _lane_total(s)
        qv = _lane_total(qs)
        mv = tv * (1.0 / d)
        var = qv * (1.0 / d) - mv * mv + EPS
        bits = lax.bitcast_convert_type(var, jnp.int32)
        y = lax.bitcast_convert_type(
            jnp.full((LANES,), 0x5F3759DF, jnp.int32) - (bits >> 1),
            jnp.float32)
        half, threehalf = var * -0.5, 1.5
        for _ in range(3):
            y = y * (threehalf + half * y * y)
        for j in range(ng):
            rows_v[b, r, pl.ds(LANES * j, LANES)] = (
                (e[j] - mv) * y * wv[j] + bv[j])


def _sc_fused(table, flat_idx, pos_flat, ln_w, ln_b):
    n, = flat_idx.shape
    _, d = table.shape
    l = pos_flat.shape[0] // d
    ng = d // LANES
    info = plsc.get_sparse_core_info()
    nw = info.num_cores * info.num_subcores
    per_w = n // nw
    chunk = 128
    nbuf = 2
    rounds = per_w // (chunk * nbuf)
    mesh = plsc.VectorSubcoreMesh(core_axis_name="c", subcore_axis_name="s")

    @functools.partial(
        pl.kernel,
        mesh=mesh,
        out_type=jax.ShapeDtypeStruct((n, d), jnp.float32),
        scratch_types=[
            pltpu.VMEM((nbuf, chunk), jnp.int32),
            pltpu.VMEM((nbuf, chunk, d), jnp.float32),
            pltpu.VMEM((l * d,), jnp.float32),
            pltpu.VMEM((d,), jnp.float32),
            pltpu.VMEM((d,), jnp.float32),
            pltpu.SemaphoreType.DMA((nbuf,)),
            pltpu.SemaphoreType.DMA((nbuf,)),
        ],
    )
    def fused_kernel(table_hbm, idx_hbm, pos_hbm, w_hbm, b_hbm, out_hbm,
                     idx_v, rows_v, pos_v, w_v, b_v, gsem, ssem):
        wid = lax.axis_index("s") * info.num_cores + lax.axis_index("c")
        w_base = wid * per_w
        pltpu.sync_copy(pos_hbm, pos_v)
        pltpu.sync_copy(w_hbm, w_v)
        pltpu.sync_copy(b_hbm, b_v)
        wv = [w_v[pl.ds(LANES * j, LANES)] for j in range(ng)]
        bv = [b_v[pl.ds(LANES * j, LANES)] for j in range(ng)]

        def compute(c, b):
            pltpu.make_async_copy(
                table_hbm.at[idx_v.at[b]], rows_v.at[b], gsem.at[b]).wait()
            base = w_base + c * chunk
            _ln_rows(rows_v, b, pos_v, wv, bv, base, chunk, l, d)
            pltpu.async_copy(
                rows_v.at[b], out_hbm.at[pl.ds(base, chunk)], ssem.at[b])

        def round_body(i, carry):
            for b in range(nbuf):
                c = i * nbuf + b

                @pl.when(i > 0)
                def _():
                    prev = w_base + ((i - 1) * nbuf + b) * chunk
                    pltpu.make_async_copy(
                        rows_v.at[b], out_hbm.at[pl.ds(prev, chunk)],
                        ssem.at[b]).wait()

                pltpu.sync_copy(
                    idx_hbm.at[pl.ds(w_base + c * chunk, chunk)], idx_v.at[b])
                pltpu.async_copy(
                    table_hbm.at[idx_v.at[b]], rows_v.at[b], gsem.at[b])
                pb = (b - 1) % nbuf
                if b == 0:
                    @pl.when(i > 0)
                    def _():
                        compute(i * nbuf - 1, pb)
                else:
                    compute(c - 1, pb)
            return carry

        lax.fori_loop(0, rounds, round_body, 0)
        compute(rounds * nbuf - 1, nbuf - 1)
        for b in range(nbuf):
            last = w_base + ((rounds - 1) * nbuf + b) * chunk
            pltpu.make_async_copy(
                rows_v.at[b], out_hbm.at[pl.ds(last, chunk)], ssem.at[b]).wait()

    return fused_kernel(table, flat_idx, pos_flat, ln_w, ln_b)


@jax.jit
def kernel(x, token_table, pos_table, ln_w, ln_b):
    b, l = x.shape
    flat = x.reshape(-1).astype(jnp.int32)
    out = _sc_fused(token_table, flat, pos_table[:l].reshape(-1), ln_w, ln_b)
    return out.reshape(b, l, -1)


# speedup vs baseline: 1.0014x; 1.0014x over previous
import functools

import jax
import jax.numpy as jnp
from jax import lax
from jax.experimental import pallas as pl
from jax.experimental.pallas import tpu as pltpu
import jax.experimental.pallas.tpu_sc as plsc

EPS = 1e-5
NSLICE = 4


def _sc_gather(table, flat_idx):
    n, = flat_idx.shape
    _, d = table.shape
    info = plsc.get_sparse_core_info()
    nw = info.num_cores * info.num_subcores
    per_w = n // nw
    chunk = 128
    nbuf = 2
    n_iters = per_w // (chunk * nbuf)
    mesh = plsc.VectorSubcoreMesh(core_axis_name="c", subcore_axis_name="s")

    @functools.partial(
        pl.kernel,
        mesh=mesh,
        out_type=jax.ShapeDtypeStruct((n, d), jnp.float32),
        scratch_types=[
            pltpu.VMEM((nbuf, chunk), jnp.int32),
            pltpu.VMEM((nbuf, chunk, d), jnp.float32),
            pltpu.SemaphoreType.DMA((nbuf,)),
            pltpu.SemaphoreType.DMA((nbuf,)),
        ],
    )
    def gather_kernel(table_hbm, idx_hbm, out_hbm, idx_v, rows_v, gsem, ssem):
        wid = lax.axis_index("s") * info.num_cores + lax.axis_index("c")
        w_base = wid * per_w

        def body(i, carry):
            for b in range(nbuf):
                base = w_base + (i * nbuf + b) * chunk

                @pl.when(i > 0)
                def _():
                    prev = w_base + ((i - 1) * nbuf + b) * chunk
                    pltpu.make_async_copy(
                        rows_v.at[b], out_hbm.at[pl.ds(prev, chunk)],
                        ssem.at[b]).wait()

                pltpu.sync_copy(idx_hbm.at[pl.ds(base, chunk)], idx_v.at[b])
                pltpu.async_copy(
                    table_hbm.at[idx_v.at[b]], rows_v.at[b], gsem.at[b])
            for b in range(nbuf):
                base = w_base + (i * nbuf + b) * chunk
                pltpu.make_async_copy(
                    table_hbm.at[idx_v.at[b]], rows_v.at[b], gsem.at[b]).wait()
                pltpu.async_copy(
                    rows_v.at[b], out_hbm.at[pl.ds(base, chunk)], ssem.at[b])
            return carry

        lax.fori_loop(0, n_iters, body, 0)
        for b in range(nbuf):
            last = w_base + ((n_iters - 1) * nbuf + b) * chunk
            pltpu.make_async_copy(
                rows_v.at[b], out_hbm.at[pl.ds(last, chunk)], ssem.at[b]).wait()

    return gather_kernel(table, flat_idx)


def _tc_pos_ln(tok, pos_table, ln_w, ln_b):
    b, l, d = tok.shape
    bb = 64
    grid = (b // bb,)

    def body(tok_ref, pos_ref, w_ref, b_ref, out_ref):
        e = tok_ref[...] + pos_ref[...][None]
        m = jnp.mean(e, axis=-1, keepdims=True)
        c = e - m
        v = jnp.mean(c * c, axis=-1, keepdims=True)
        out_ref[...] = (c * lax.rsqrt(v + EPS)) * w_ref[...] + b_ref[...]

    return pl.pallas_call(
        body,
        grid=grid,
        in_specs=[
            pl.BlockSpec((bb, l, d), lambda i: (i, 0, 0)),
            pl.BlockSpec((l, d), lambda i: (0, 0)),
            pl.BlockSpec((d,), lambda i: (0,)),
            pl.BlockSpec((d,), lambda i: (0,)),
        ],
        out_specs=pl.BlockSpec((bb, l, d), lambda i: (i, 0, 0)),
        out_shape=jax.ShapeDtypeStruct((b, l, d), jnp.float32),
    )(tok, pos_table, ln_w, ln_b)


@jax.jit
def kernel(x, token_table, pos_table, ln_w, ln_b):
    b, l = x.shape
    x = x.astype(jnp.int32)
    pos = pos_table[:l]
    bs = b // NSLICE
    outs = []
    for k in range(NSLICE):
        flat = x[k * bs:(k + 1) * bs].reshape(-1)
        tok = _sc_gather(token_table, flat)
        outs.append(_tc_pos_ln(tok.reshape(bs, l, -1), pos, ln_w, ln_b))
    return jnp.concatenate(outs, axis=0)


# speedup vs baseline: 1.5324x; 1.5302x over previous
import functools

import jax
import jax.numpy as jnp
from jax import lax
from jax.experimental import pallas as pl
from jax.experimental.pallas import tpu as pltpu
import jax.experimental.pallas.tpu_sc as plsc

EPS = 1e-5


def _sc_gather(table, flat_idx):
    n, = flat_idx.shape
    _, d = table.shape
    info = plsc.get_sparse_core_info()
    nw = info.num_cores * info.num_subcores
    per_w = n // nw
    chunk = 128
    nbuf = 4
    n_iters = per_w // (chunk * nbuf)
    mesh = plsc.VectorSubcoreMesh(core_axis_name="c", subcore_axis_name="s")

    @functools.partial(
        pl.kernel,
        mesh=mesh,
        out_type=jax.ShapeDtypeStruct((n, d), jnp.float32),
        scratch_types=[
            pltpu.VMEM((nbuf, chunk), jnp.int32),
            pltpu.VMEM((nbuf, chunk, d), jnp.float32),
            pltpu.SemaphoreType.DMA((nbuf,)),
            pltpu.SemaphoreType.DMA((nbuf,)),
        ],
    )
    def gather_kernel(table_hbm, idx_hbm, out_hbm, idx_v, rows_v, gsem, ssem):
        wid = lax.axis_index("s") * info.num_cores + lax.axis_index("c")
        w_base = wid * per_w

        def body(i, carry):
            for b in range(nbuf):
                base = w_base + (i * nbuf + b) * chunk

                @pl.when(i > 0)
                def _():
                    prev = w_base + ((i - 1) * nbuf + b) * chunk
                    pltpu.make_async_copy(
                        rows_v.at[b], out_hbm.at[pl.ds(prev, chunk)],
                        ssem.at[b]).wait()

                pltpu.sync_copy(idx_hbm.at[pl.ds(base, chunk)], idx_v.at[b])
                pltpu.async_copy(
                    table_hbm.at[idx_v.at[b]], rows_v.at[b], gsem.at[b])
            for b in range(nbuf):
                base = w_base + (i * nbuf + b) * chunk
                pltpu.make_async_copy(
                    table_hbm.at[idx_v.at[b]], rows_v.at[b], gsem.at[b]).wait()
                pltpu.async_copy(
                    rows_v.at[b], out_hbm.at[pl.ds(base, chunk)], ssem.at[b])
            return carry

        lax.fori_loop(0, n_iters, body, 0)
        for b in range(nbuf):
            last = w_base + ((n_iters - 1) * nbuf + b) * chunk
            pltpu.make_async_copy(
                rows_v.at[b], out_hbm.at[pl.ds(last, chunk)], ssem.at[b]).wait()

    return gather_kernel(table, flat_idx)


def _tc_pos_ln(tok, pos_table, ln_w, ln_b):
    b, l, d = tok.shape
    bb = 64
    grid = (b // bb,)

    def body(tok_ref, pos_ref, w_ref, b_ref, out_ref):
        e = tok_ref[...] + pos_ref[...][None]
        m = jnp.mean(e, axis=-1, keepdims=True)
        c = e - m
        v = jnp.mean(c * c, axis=-1, keepdims=True)
        out_ref[...] = (c * lax.rsqrt(v + EPS)) * w_ref[...] + b_ref[...]

    return pl.pallas_call(
        body,
        grid=grid,
        in_specs=[
            pl.BlockSpec((bb, l, d), lambda i: (i, 0, 0)),
            pl.BlockSpec((l, d), lambda i: (0, 0)),
            pl.BlockSpec((d,), lambda i: (0,)),
            pl.BlockSpec((d,), lambda i: (0,)),
        ],
        out_specs=pl.BlockSpec((bb, l, d), lambda i: (i, 0, 0)),
        out_shape=jax.ShapeDtypeStruct((b, l, d), jnp.float32),
    )(tok, pos_table, ln_w, ln_b)


@jax.jit
def kernel(x, token_table, pos_table, ln_w, ln_b):
    b, l = x.shape
    flat = x.reshape(-1).astype(jnp.int32)
    pos = pos_table[:l]
    tok = _sc_gather(token_table, flat)
    return _tc_pos_ln(tok.reshape(b, l, -1), pos, ln_w, ln_b)
